# baseline (speedup 1.0000x reference)
import math

import jax
import jax.numpy as jnp
from jax import lax
from jax.experimental import pallas as pl
from jax.experimental.pallas import tpu as pltpu
from jax.experimental.pallas import tpu_sc as plsc

VOCAB = 1000000
DIM = 32
RANK = 8
SCALE = 1.0

NUM_CORES = 2
NUM_SUBCORES = 16
NW = NUM_CORES * NUM_SUBCORES
NB = 16384
NL = 50
N_TOK = NB * NL
TOK_PER_W = N_TOK // NW
C = 800
N_CHUNK = TOK_PER_W // C

W_PIECE = 2048
W_VB = 4 * W_PIECE


def _merge_detile_body(wt_ref, at_ref, lb_ref, y_ref):
    wt = wt_ref[...]
    at = at_ref[...]
    lb = lb_ref[...] * SCALE
    eye = jnp.eye(DIM, dtype=jnp.float32)
    pieces = []
    for j in range(4):
        wj = wt[:, j * W_PIECE:(j + 1) * W_PIECE]
        aj = at[:, j * W_PIECE:(j + 1) * W_PIECE]
        wjt = jax.lax.dot_general(
            wj, eye, dimension_numbers=(((0,), (0,)), ((), ())),
            preferred_element_type=jnp.float32)
        dj = jax.lax.dot_general(
            aj, lb, dimension_numbers=(((0,), (0,)), ((), ())),
            preferred_element_type=jnp.float32)
        pieces.append(wjt + dj)
    y_ref[...] = jnp.concatenate(pieces, axis=1)


def _merge_detile(wt, at, lb):
    grid = math.ceil(VOCAB / W_VB)
    return pl.pallas_call(
        _merge_detile_body,
        grid=(grid,),
        in_specs=[
            pl.BlockSpec((DIM, W_VB), lambda i: (0, i)),
            pl.BlockSpec((RANK, W_VB), lambda i: (0, i)),
            pl.BlockSpec((RANK, DIM), lambda i: (0, 0)),
        ],
        out_specs=pl.BlockSpec((W_PIECE, 128), lambda i: (i, 0)),
        out_shape=jax.ShapeDtypeStruct((grid * W_PIECE, 128), jnp.float32),
    )(wt, at, lb)


def _retile_body(x_ref, y_ref):
    x = x_ref[...]
    eye = jnp.eye(DIM, dtype=jnp.float32)
    pieces = [
        jax.lax.dot_general(
            eye, x[:, j * DIM:(j + 1) * DIM],
            dimension_numbers=(((1,), (1,)), ((), ())),
            preferred_element_type=jnp.float32)
        for j in range(4)
    ]
    y_ref[...] = jnp.concatenate(pieces, axis=1)


def _retile(rows128):
    return pl.pallas_call(
        _retile_body,
        grid=(NL,),
        in_specs=[pl.BlockSpec((NB // 4, 128), lambda l: (l, 0))],
        out_specs=pl.BlockSpec((DIM, NB), lambda l: (l, 0)),
        out_shape=jax.ShapeDtypeStruct((NL * DIM, NB), jnp.float32),
    )(rows128)


def _gather_body(table_hbm, idx_hbm, out_hbm,
                 idx_v0, idx_v1, sw_v0, sw_v1, so_v0, so_v1, w_v0, w_v1,
                 sem_w0, sem_w1, sem_o0, sem_o1):
    wid = lax.axis_index("s") * NUM_CORES + lax.axis_index("c")
    base = wid * TOK_PER_W

    lane = lax.iota(jnp.int32, 16)

    bufs = ((idx_v0, sw_v0, so_v0, w_v0, sem_w0, sem_o0),
            (idx_v1, sw_v1, so_v1, w_v1, sem_w1, sem_o1))

    def start_gathers(ci, b):
        idx_v, sw_v, so_v, w_v, sem_w, _ = bufs[b]
        tok = pl.multiple_of(base + ci * C, C)
        pltpu.sync_copy(idx_hbm.at[pl.ds(tok, C)], idx_v)

        def perm(g):
            off = pl.multiple_of(g * 16, 16)
            v = idx_v[pl.ds(off, 16)]
            rw = v & (W_VB - 1)
            sw_v[pl.ds(off, 16)] = (
                (v - rw) + ((rw & (W_PIECE - 1)) << 2) + (rw >> 11))
            t = tok + off + lane
            bb = t // NL
            l = t - bb * NL
            so_v[pl.ds(off, 16)] = (
                l * NB + ((bb & 4095) << 2) + (bb >> 12))

        plsc.parallel_loop(0, C // 16, 1, unroll=4, carry=None)(perm)
        pltpu.make_async_copy(table_hbm.at[sw_v], w_v, sem_w).start()

    def wait_gathers(b):
        _, sw_v, _, w_v, sem_w, _ = bufs[b]
        pltpu.make_async_copy(table_hbm.at[sw_v], w_v, sem_w).wait()

    def start_write(b):
        _, _, so_v, w_v, _, sem_o = bufs[b]
        pltpu.make_async_copy(w_v, out_hbm.at[so_v], sem_o).start()

    def wait_write(b):
        _, _, so_v, w_v, _, sem_o = bufs[b]
        pltpu.make_async_copy(w_v, out_hbm.at[so_v], sem_o).wait()

    start_gathers(0, 0)

    def outer(cj, _):
        for b in (0, 1):
            ci = cj * 2 + b
            wait_gathers(b)
            nb = 1 - b

            @pl.when(ci >= 1)
            def _():
                wait_write(nb)

            @pl.when(ci + 1 < N_CHUNK)
            def _():
                start_gathers(ci + 1, nb)

            start_write(b)
        return 0

    lax.fori_loop(0, N_CHUNK // 2, outer, 0)
    wait_write(1)


@jax.jit
def _lora_embed(table, idx):
    mesh = plsc.VectorSubcoreMesh(core_axis_name="c", subcore_axis_name="s")
    fn = pl.kernel(
        _gather_body,
        out_type=jax.ShapeDtypeStruct((N_TOK, DIM), jnp.float32),
        mesh=mesh,
        compiler_params=pltpu.CompilerParams(
            needs_layout_passes=False, use_tc_tiling_on_sc=False),
        scratch_types=[
            pltpu.VMEM((C,), jnp.int32),
            pltpu.VMEM((C,), jnp.int32),
            pltpu.VMEM((C,), jnp.int32),
            pltpu.VMEM((C,), jnp.int32),
            pltpu.VMEM((C,), jnp.int32),
            pltpu.VMEM((C,), jnp.int32),
            pltpu.VMEM((C, DIM), jnp.float32),
            pltpu.VMEM((C, DIM), jnp.float32),
            pltpu.SemaphoreType.DMA,
            pltpu.SemaphoreType.DMA,
            pltpu.SemaphoreType.DMA,
            pltpu.SemaphoreType.DMA,
        ],
    )
    return fn(table, idx)


def kernel(x, weight, lora_A, lora_B):
    idx = x.reshape(-1).astype(jnp.int32)
    merged = _merge_detile(weight.T, lora_A.T, lora_B)
    table = merged.reshape(-1, DIM)
    rows = _lora_embed(table, idx)
    y2 = _retile(rows.reshape(N_TOK // 4, 128))
    return jnp.transpose(y2.reshape(NL, DIM, NB), (2, 0, 1))


# speedup vs baseline: 1.0881x; 1.0881x over previous
import math

import jax
import jax.numpy as jnp
from jax import lax
from jax.experimental import pallas as pl
from jax.experimental.pallas import tpu as pltpu
from jax.experimental.pallas import tpu_sc as plsc

VOCAB = 1000000
DIM = 32
RANK = 8
SCALE = 1.0

NUM_CORES = 2
NUM_SUBCORES = 16
NW = NUM_CORES * NUM_SUBCORES
NB = 16384
NL = 50
N_TOK = NB * NL
TOK_PER_W = N_TOK // NW
C = 1600
N_CHUNK = TOK_PER_W // C

W_PIECE = 8192
W_SHIFT = 13
W_VB = 4 * W_PIECE
L_PER = 2


def _merge_detile_body(wt_ref, at_ref, lb_ref, y_ref):
    m = wt_ref[...] + jax.lax.dot_general(
        lb_ref[...], at_ref[...],
        dimension_numbers=(((0,), (0,)), ((), ())),
        preferred_element_type=jnp.float32) * SCALE
    pieces = [jnp.transpose(m[:, j * W_PIECE:(j + 1) * W_PIECE])
              for j in range(4)]
    y_ref[...] = jnp.concatenate(pieces, axis=1)


def _merge_detile(wt, at, lb):
    grid = math.ceil(VOCAB / W_VB)
    return pl.pallas_call(
        _merge_detile_body,
        grid=(grid,),
        in_specs=[
            pl.BlockSpec((DIM, W_VB), lambda i: (0, i)),
            pl.BlockSpec((RANK, W_VB), lambda i: (0, i)),
            pl.BlockSpec((RANK, DIM), lambda i: (0, 0)),
        ],
        out_specs=pl.BlockSpec((W_PIECE, 128), lambda i: (i, 0)),
        out_shape=jax.ShapeDtypeStruct((grid * W_PIECE, 128), jnp.float32),
    )(wt, at, lb)


def _retile_body(x_ref, y_ref):
    x = x_ref[...]
    groups = []
    for g in range(L_PER):
        xg = x[g * (NB // 4):(g + 1) * (NB // 4)]
        pieces = [jnp.transpose(xg[:, j * DIM:(j + 1) * DIM])
                  for j in range(4)]
        groups.append(jnp.concatenate(pieces, axis=1))
    y_ref[...] = jnp.concatenate(groups, axis=0)


def _retile(rows128):
    return pl.pallas_call(
        _retile_body,
        grid=(NL // L_PER,),
        in_specs=[pl.BlockSpec((L_PER * NB // 4, 128), lambda l: (l, 0))],
        out_specs=pl.BlockSpec((L_PER * DIM, NB), lambda l: (l, 0)),
        out_shape=jax.ShapeDtypeStruct((NL * DIM, NB), jnp.float32),
    )(rows128)


def _gather_body(table_hbm, idx_hbm, out_hbm,
                 idx_v0, idx_v1, sw_v0, sw_v1, so_v0, so_v1, w_v0, w_v1,
                 sem_w0, sem_w1, sem_o0, sem_o1):
    wid = lax.axis_index("s") * NUM_CORES + lax.axis_index("c")
    base = wid * TOK_PER_W

    lane = lax.iota(jnp.int32, 16)

    bufs = ((idx_v0, sw_v0, so_v0, w_v0, sem_w0, sem_o0),
            (idx_v1, sw_v1, so_v1, w_v1, sem_w1, sem_o1))

    def start_gathers(ci, b):
        idx_v, sw_v, so_v, w_v, sem_w, _ = bufs[b]
        tok = pl.multiple_of(base + ci * C, C)
        pltpu.sync_copy(idx_hbm.at[pl.ds(tok, C)], idx_v)

        def perm(g):
            off = pl.multiple_of(g * 16, 16)
            v = idx_v[pl.ds(off, 16)]
            rw = v & (W_VB - 1)
            sw_v[pl.ds(off, 16)] = (
                (v - rw) + ((rw & (W_PIECE - 1)) << 2) + (rw >> W_SHIFT))
            t = tok + off + lane
            bb = t // NL
            l = t - bb * NL
            so_v[pl.ds(off, 16)] = (
                l * NB + ((bb & 4095) << 2) + (bb >> 12))

        plsc.parallel_loop(0, C // 16, 1, unroll=4, carry=None)(perm)
        pltpu.make_async_copy(table_hbm.at[sw_v], w_v, sem_w).start()

    def wait_gathers(b):
        _, sw_v, _, w_v, sem_w, _ = bufs[b]
        pltpu.make_async_copy(table_hbm.at[sw_v], w_v, sem_w).wait()

    def start_write(b):
        _, _, so_v, w_v, _, sem_o = bufs[b]
        pltpu.make_async_copy(w_v, out_hbm.at[so_v], sem_o).start()

    def wait_write(b):
        _, _, so_v, w_v, _, sem_o = bufs[b]
        pltpu.make_async_copy(w_v, out_hbm.at[so_v], sem_o).wait()

    start_gathers(0, 0)

    def outer(cj, _):
        for b in (0, 1):
            ci = cj * 2 + b
            wait_gathers(b)
            nb = 1 - b

            @pl.when(ci >= 1)
            def _():
                wait_write(nb)

            @pl.when(ci + 1 < N_CHUNK)
            def _():
                start_gathers(ci + 1, nb)

            start_write(b)
        return 0

    lax.fori_loop(0, N_CHUNK // 2, outer, 0)
    wait_write(1)


@jax.jit
def _lora_embed(table, idx):
    mesh = plsc.VectorSubcoreMesh(core_axis_name="c", subcore_axis_name="s")
    fn = pl.kernel(
        _gather_body,
        out_type=jax.ShapeDtypeStruct((N_TOK, DIM), jnp.float32),
        mesh=mesh,
        compiler_params=pltpu.CompilerParams(
            needs_layout_passes=False, use_tc_tiling_on_sc=False),
        scratch_types=[
            pltpu.VMEM((C,), jnp.int32),
            pltpu.VMEM((C,), jnp.int32),
            pltpu.VMEM((C,), jnp.int32),
            pltpu.VMEM((C,), jnp.int32),
            pltpu.VMEM((C,), jnp.int32),
            pltpu.VMEM((C,), jnp.int32),
            pltpu.VMEM((C, DIM), jnp.float32),
            pltpu.VMEM((C, DIM), jnp.float32),
            pltpu.SemaphoreType.DMA,
            pltpu.SemaphoreType.DMA,
            pltpu.SemaphoreType.DMA,
            pltpu.SemaphoreType.DMA,
        ],
    )
    return fn(table, idx)


def kernel(x, weight, lora_A, lora_B):
    idx = x.reshape(-1).astype(jnp.int32)
    merged = _merge_detile(weight.T, lora_A.T, lora_B)
    table = merged.reshape(-1, DIM)
    rows = _lora_embed(table, idx)
    y2 = _retile(rows.reshape(N_TOK // 4, 128))
    return jnp.transpose(y2.reshape(NL, DIM, NB), (2, 0, 1))


# speedup vs baseline: 1.3412x; 1.2325x over previous
import math

import jax
import jax.numpy as jnp
from jax import lax
from jax.experimental import pallas as pl
from jax.experimental.pallas import tpu as pltpu
from jax.experimental.pallas import tpu_sc as plsc

VOCAB = 1000000
DIM = 32
RANK = 8
SCALE = 1.0

NUM_CORES = 2
NUM_SUBCORES = 16
NW = NUM_CORES * NUM_SUBCORES
NB = 16384
NL = 50
N_TOK = NB * NL
TOK_PER_W = N_TOK // NW
C = 1600
N_CHUNK = TOK_PER_W // C

W_PIECE = 8192
W_SHIFT = 13
W_VB = 4 * W_PIECE
L_PER = 2


def _merge_detile_body(wt_ref, at_ref, lb_ref, y_ref):
    m = wt_ref[...] + jax.lax.dot_general(
        lb_ref[...], at_ref[...],
        dimension_numbers=(((0,), (0,)), ((), ())),
        preferred_element_type=jnp.float32) * SCALE
    pieces = [jnp.transpose(m[:, j * W_PIECE:(j + 1) * W_PIECE])
              for j in range(4)]
    y_ref[...] = jnp.concatenate(pieces, axis=1)


def _merge_detile(wt, at, lb):
    grid = math.ceil(VOCAB / W_VB)
    return pl.pallas_call(
        _merge_detile_body,
        grid=(grid,),
        in_specs=[
            pl.BlockSpec((DIM, W_VB), lambda i: (0, i)),
            pl.BlockSpec((RANK, W_VB), lambda i: (0, i)),
            pl.BlockSpec((RANK, DIM), lambda i: (0, 0)),
        ],
        out_specs=pl.BlockSpec((W_PIECE, 128), lambda i: (i, 0)),
        out_shape=jax.ShapeDtypeStruct((grid * W_PIECE, 128), jnp.float32),
    )(wt, at, lb)


def _retile_body(x_ref, y_ref):
    x = x_ref[...]
    eye = jnp.eye(DIM, dtype=jnp.float32)
    groups = []
    for g in range(L_PER):
        xg = x[g * (NB // 4):(g + 1) * (NB // 4)]
        pieces = [
            jax.lax.dot_general(
                eye, xg[:, j * DIM:(j + 1) * DIM],
                dimension_numbers=(((1,), (1,)), ((), ())),
                preferred_element_type=jnp.float32)
            for j in range(4)
        ]
        groups.append(jnp.concatenate(pieces, axis=1))
    y_ref[...] = jnp.concatenate(groups, axis=0)


def _retile(rows128):
    return pl.pallas_call(
        _retile_body,
        grid=(NL // L_PER,),
        in_specs=[pl.BlockSpec((L_PER * NB // 4, 128), lambda l: (l, 0))],
        out_specs=pl.BlockSpec((L_PER * DIM, NB), lambda l: (l, 0)),
        out_shape=jax.ShapeDtypeStruct((NL * DIM, NB), jnp.float32),
    )(rows128)


def _gather_body(table_hbm, idx_hbm, out_hbm,
                 idx_v0, idx_v1, sw_v0, sw_v1, so_v0, so_v1, w_v0, w_v1,
                 sem_w0, sem_w1, sem_o0, sem_o1):
    wid = lax.axis_index("s") * NUM_CORES + lax.axis_index("c")
    base = wid * TOK_PER_W

    lane = lax.iota(jnp.int32, 16)

    bufs = ((idx_v0, sw_v0, so_v0, w_v0, sem_w0, sem_o0),
            (idx_v1, sw_v1, so_v1, w_v1, sem_w1, sem_o1))

    def start_gathers(ci, b):
        idx_v, sw_v, so_v, w_v, sem_w, _ = bufs[b]
        tok = pl.multiple_of(base + ci * C, C)
        pltpu.sync_copy(idx_hbm.at[pl.ds(tok, C)], idx_v)

        def perm(g):
            off = pl.multiple_of(g * 16, 16)
            v = idx_v[pl.ds(off, 16)]
            rw = v & (W_VB - 1)
            sw_v[pl.ds(off, 16)] = (
                (v - rw) + ((rw & (W_PIECE - 1)) << 2) + (rw >> W_SHIFT))
            t = tok + off + lane
            bb = t // NL
            l = t - bb * NL
            so_v[pl.ds(off, 16)] = (
                l * NB + ((bb & 4095) << 2) + (bb >> 12))

        plsc.parallel_loop(0, C // 16, 1, unroll=4, carry=None)(perm)
        pltpu.make_async_copy(table_hbm.at[sw_v], w_v, sem_w).start()

    def wait_gathers(b):
        _, sw_v, _, w_v, sem_w, _ = bufs[b]
        pltpu.make_async_copy(table_hbm.at[sw_v], w_v, sem_w).wait()

    def start_write(b):
        _, _, so_v, w_v, _, sem_o = bufs[b]
        pltpu.make_async_copy(w_v, out_hbm.at[so_v], sem_o).start()

    def wait_write(b):
        _, _, so_v, w_v, _, sem_o = bufs[b]
        pltpu.make_async_copy(w_v, out_hbm.at[so_v], sem_o).wait()

    start_gathers(0, 0)

    def outer(cj, _):
        for b in (0, 1):
            ci = cj * 2 + b
            wait_gathers(b)
            nb = 1 - b

            @pl.when(ci >= 1)
            def _():
                wait_write(nb)

            @pl.when(ci + 1 < N_CHUNK)
            def _():
                start_gathers(ci + 1, nb)

            start_write(b)
        return 0

    lax.fori_loop(0, N_CHUNK // 2, outer, 0)
    wait_write(1)


@jax.jit
def _lora_embed(table, idx):
    mesh = plsc.VectorSubcoreMesh(core_axis_name="c", subcore_axis_name="s")
    fn = pl.kernel(
        _gather_body,
        out_type=jax.ShapeDtypeStruct((N_TOK, DIM), jnp.float32),
        mesh=mesh,
        compiler_params=pltpu.CompilerParams(
            needs_layout_passes=False, use_tc_tiling_on_sc=False),
        scratch_types=[
            pltpu.VMEM((C,), jnp.int32),
            pltpu.VMEM((C,), jnp.int32),
            pltpu.VMEM((C,), jnp.int32),
            pltpu.VMEM((C,), jnp.int32),
            pltpu.VMEM((C,), jnp.int32),
            pltpu.VMEM((C,), jnp.int32),
            pltpu.VMEM((C, DIM), jnp.float32),
            pltpu.VMEM((C, DIM), jnp.float32),
            pltpu.SemaphoreType.DMA,
            pltpu.SemaphoreType.DMA,
            pltpu.SemaphoreType.DMA,
            pltpu.SemaphoreType.DMA,
        ],
    )
    return fn(table, idx)


def kernel(x, weight, lora_A, lora_B):
    idx = x.reshape(-1).astype(jnp.int32)
    merged = _merge_detile(weight.T, lora_A.T, lora_B)
    table = merged.reshape(-1, DIM)
    rows = _lora_embed(table, idx)
    y2 = _retile(rows.reshape(N_TOK // 4, 128))
    return jnp.transpose(y2.reshape(NL, DIM, NB), (2, 0, 1))
